# baseline (speedup 1.0000x reference)
import functools

import jax
import jax.numpy as jnp
from jax import lax
from jax.experimental import pallas as pl
from jax.experimental.pallas import tpu as pltpu
from jax.experimental.pallas import tpu_sc as plsc

N = 16384
D = 64
LANES = 16
CHUNK = 128


def _make_kernel():
    info = plsc.get_sparse_core_info()
    nc, ns = info.num_cores, info.num_subcores
    nw = nc * ns
    b_per_w = N // nw
    n_chunks = b_per_w // CHUNK
    n_groups = b_per_w // LANES
    mesh = plsc.VectorSubcoreMesh(core_axis_name="c", subcore_axis_name="s")

    @functools.partial(
        pl.kernel,
        mesh=mesh,
        out_type=jax.ShapeDtypeStruct((N // 2, 2 * D), jnp.float32),
        scratch_types=[
            pltpu.VMEM((b_per_w,), jnp.int32),
            pltpu.VMEM((b_per_w,), jnp.int32),
            pltpu.VMEM((b_per_w, 2 * D), jnp.float32),
            pltpu.VMEM((b_per_w // 2, 2 * D), jnp.float32),
            pltpu.SemaphoreType.DMA,
        ],
        compiler_params=pltpu.CompilerParams(use_tc_tiling_on_sc=True,
                                             needs_layout_passes=False),
    )
    def emb_kernel(tok_hbm, tw_hbm, pos_hbm, out_hbm,
                   tok_v, gidx_v, rows_v, out_v, sem):
        wid = lax.axis_index("s") * nc + lax.axis_index("c")
        base = pl.multiple_of(wid * b_per_w, b_per_w)
        base2 = pl.multiple_of(wid * (b_per_w // 2), b_per_w // 2)

        pltpu.sync_copy(tok_hbm.at[pl.ds(base, b_per_w)], tok_v)

        def shift(i, c):
            sl = pl.ds(i * LANES, LANES)
            gidx_v[sl] = tok_v[sl] >> 1
            return c
        lax.fori_loop(0, n_groups, shift, 0, unroll=4)

        copies = [
            pltpu.async_copy(
                tw_hbm.at[gidx_v.at[pl.ds(k * CHUNK, CHUNK)]],
                rows_v.at[pl.ds(k * CHUNK, CHUNK)],
                sem)
            for k in range(n_chunks)
        ]
        pltpu.sync_copy(pos_hbm.at[pl.ds(base2, b_per_w // 2)], out_v)
        for c in copies:
            c.wait()

        iota = lax.iota(jnp.int32, LANES)

        def compact(g, c):
            sl = pl.ds(g * LANES, LANES)
            tvec = tok_v[sl]
            par = (tvec & 1) * D
            rvec = g * LANES + iota
            orow = rvec >> 1
            ocol = (rvec & 1) * D

            def feat(d, c2):
                val = plsc.load_gather(rows_v, [rvec, par + d])
                plsc.addupdate_scatter(out_v, [orow, ocol + d], val)
                return c2
            lax.fori_loop(0, D, feat, 0, unroll=4)
            return c
        lax.fori_loop(0, n_groups, compact, 0)

        pltpu.sync_copy(out_v, out_hbm.at[pl.ds(base2, b_per_w // 2)])

    return emb_kernel


_emb = _make_kernel()


def kernel(tokens, token_weight, pos_weight):
    out = _emb(tokens.astype(jnp.int32),
               token_weight.reshape(-1, 2 * D),
               pos_weight.reshape(-1, 2 * D))
    return out.reshape(N, D)
